# baseline (speedup 1.0000x reference)
import functools

import jax
import jax.numpy as jnp
from jax import lax
from jax.experimental import pallas as pl
from jax.experimental.pallas import tpu as pltpu
from jax.experimental.pallas import tpu_sc as plsc

D = 256
NH = 8
NL = 4
NP = 4
HD = D // NH
SPATIAL = ((64, 64), (32, 32), (16, 16), (8, 8))
STARTS = (0, 4096, 5120, 5376)
LEN_IN = 5440
B = 2
LQ = 5440
ROWS = B * LQ
SBLK = 640
GRID = ROWS // SBLK
NWORK = 32
RW = ROWS // NWORK
CH = 5
LANES = NH * NL * NP
NSLAB_I = 2
NSLAB_W = 4


def _proj_kernel(q_ref, inp_ref, refx_ref, refy_ref,
                 wx_ref, wy_ref, waw_ref, wv_ref,
                 bx_ref, by_ref, baw_ref, bv_ref,
                 val_ref, pig_ref, pil_ref, pwg_ref, pwl_ref):
    i = pl.program_id(0)
    q = q_ref[...]

    val = lax.dot_general(
        inp_ref[...], wv_ref[...], (((1,), (1,)), ((), ())),
        preferred_element_type=jnp.float32) + bv_ref[...]
    val_ref[...] = val.astype(jnp.bfloat16)

    sox = lax.dot_general(q, wx_ref[...], (((1,), (1,)), ((), ())),
                          preferred_element_type=jnp.float32) + bx_ref[...]
    soy = lax.dot_general(q, wy_ref[...], (((1,), (1,)), ((), ())),
                          preferred_element_type=jnp.float32) + by_ref[...]

    logit = lax.dot_general(q, waw_ref[...], (((1,), (1,)), ((), ())),
                            preferred_element_type=jnp.float32) + baw_ref[...]
    m = jnp.max(logit, axis=1, keepdims=True)
    e = jnp.exp(logit - m)
    si = lax.broadcasted_iota(jnp.int32, (LANES, LANES), 0)
    sj = lax.broadcasted_iota(jnp.int32, (LANES, LANES), 1)
    seg = ((si >> 4) == (sj >> 4)).astype(jnp.float32)
    denom = lax.dot_general(e, seg, (((1,), (0,)), ((), ())),
                            preferred_element_type=jnp.float32,
                            precision=lax.Precision.HIGHEST)
    aw = e / denom

    li = lax.broadcasted_iota(jnp.int32, (NL, LANES), 0)
    lj = lax.broadcasted_iota(jnp.int32, (NL, LANES), 1)
    exp_mat = (((lj >> 2) & 3) == li).astype(jnp.float32)
    refx = lax.dot_general(refx_ref[...], exp_mat, (((1,), (0,)), ((), ())),
                           preferred_element_type=jnp.float32,
                           precision=lax.Precision.HIGHEST)
    refy = lax.dot_general(refy_ref[...], exp_mat, (((1,), (0,)), ((), ())),
                           preferred_element_type=jnp.float32,
                           precision=lax.Precision.HIGHEST)

    lane = lax.broadcasted_iota(jnp.int32, (SBLK, LANES), 1)
    lvl = (lane >> 2) & 3
    h_lane = lane >> 4

    wi = jnp.full((SBLK, LANES), SPATIAL[0][1], jnp.int32)
    hi = jnp.full((SBLK, LANES), SPATIAL[0][0], jnp.int32)
    st = jnp.full((SBLK, LANES), STARTS[0], jnp.int32)
    for l in range(1, NL):
        wi = jnp.where(lvl == l, SPATIAL[l][1], wi)
        hi = jnp.where(lvl == l, SPATIAL[l][0], hi)
        st = jnp.where(lvl == l, STARTS[l], st)
    wf = wi.astype(jnp.float32)
    hf = hi.astype(jnp.float32)

    x = jnp.clip(refx + sox, 0.0, 1.0) * wf - 0.5
    y = jnp.clip(refy + soy, 0.0, 1.0) * hf - 0.5

    flx = jnp.floor(x).astype(jnp.int32)
    fly = jnp.floor(y).astype(jnp.int32)
    x0 = jnp.clip(flx, 0, wi - 1)
    x1 = jnp.clip(flx + 1, 0, wi - 1)
    y0 = jnp.clip(fly, 0, hi - 1)
    y1 = jnp.clip(fly + 1, 0, hi - 1)
    xa = jnp.clip(flx, 0, wi - 2)
    x0f = x0.astype(jnp.float32)
    x1f = x1.astype(jnp.float32)
    y0f = y0.astype(jnp.float32)
    y1f = y1.astype(jnp.float32)

    wa = aw * ((x1f - x) * (y1f - y))
    wb = aw * ((x1f - x) * (y - y0f))
    wc = aw * ((x - x0f) * (y1f - y))
    wd = aw * ((x - x0f) * (y - y0f))

    zero = jnp.zeros_like(wa)
    sel0 = x0 == xa
    sel1 = x1 == xa
    wl0 = jnp.where(sel0, wa, zero) + jnp.where(sel1, wc, zero)
    wr0 = jnp.where(sel0, zero, wa) + jnp.where(sel1, zero, wc)
    wl1 = jnp.where(sel0, wb, zero) + jnp.where(sel1, wd, zero)
    wr1 = jnp.where(sel0, zero, wb) + jnp.where(sel1, zero, wd)

    row0 = i * SBLK
    ridx = row0 + lax.broadcasted_iota(jnp.int32, (SBLK, LANES), 0)
    base = (ridx // LQ) * (LEN_IN * NH)

    iy0 = base + (st + y0 * wi + xa) * NH + h_lane
    iy1 = base + (st + y1 * wi + xa) * NH + h_lane
    iy0l = ((st - 5120) + y0 * wi + xa) * NH + h_lane
    iy1l = ((st - 5120) + y1 * wi + xa) * NH + h_lane

    ci = lax.broadcasted_iota(jnp.int32, (LANES, LANES // 2), 0)
    cj = lax.broadcasted_iota(jnp.int32, (LANES, LANES // 2), 1)
    clvl = (ci >> 2) & 3
    jg = (ci >> 4) * 8 + ((ci >> 2) & 1) * 4 + (ci & 3)
    jl = (ci >> 4) * 8 + (((ci >> 2) & 3) - 2) * 4 + (ci & 3)
    Pg = ((jg == cj) & (clvl < 2)).astype(jnp.float32)
    Pl = ((jl == cj) & (clvl >= 2)).astype(jnp.float32)

    fi = lax.broadcasted_iota(jnp.int32, (LANES, LANES), 0)
    fj = lax.broadcasted_iota(jnp.int32, (LANES, LANES), 1)
    flvl = (fi >> 2) & 3
    fjg = (fi >> 4) * 8 + ((fi >> 2) & 1) * 4 + (fi & 3)
    fjl = (fi >> 4) * 8 + (((fi >> 2) & 3) - 2) * 4 + (fi & 3)
    PgE0 = ((fj == 2 * fjg) & (flvl < 2)).astype(jnp.float32)
    PgE1 = ((fj == 2 * fjg + 1) & (flvl < 2)).astype(jnp.float32)
    PlE0 = ((fj == 2 * fjl) & (flvl >= 2)).astype(jnp.float32)
    PlE1 = ((fj == 2 * fjl + 1) & (flvl >= 2)).astype(jnp.float32)

    def compact(arr, P):
        return lax.dot_general(arr, P, (((1,), (0,)), ((), ())),
                               preferred_element_type=jnp.float32,
                               precision=lax.Precision.HIGHEST)

    def compact_i(arr, P):
        return compact(arr.astype(jnp.float32), P).astype(jnp.int32)

    H = LANES // 2
    pig_ref[:, 0 * H:1 * H] = compact_i(iy0, Pg)
    pig_ref[:, 1 * H:2 * H] = compact_i(iy0 + NH, Pg)
    pig_ref[:, 2 * H:3 * H] = compact_i(iy1, Pg)
    pig_ref[:, 3 * H:4 * H] = compact_i(iy1 + NH, Pg)
    pil_ref[:, 0 * H:1 * H] = compact_i(iy0l, Pl)
    pil_ref[:, 1 * H:2 * H] = compact_i(iy0l + NH, Pl)
    pil_ref[:, 2 * H:3 * H] = compact_i(iy1l, Pl)
    pil_ref[:, 3 * H:4 * H] = compact_i(iy1l + NH, Pl)
    pwg_ref[:, 0 * LANES:1 * LANES] = (
        compact(wl0, PgE0) + compact(wr0, PgE1)).astype(jnp.bfloat16)
    pwg_ref[:, 1 * LANES:2 * LANES] = (
        compact(wl1, PgE0) + compact(wr1, PgE1)).astype(jnp.bfloat16)
    pwl_ref[:, 0 * LANES:1 * LANES] = (
        compact(wl0, PlE0) + compact(wr0, PlE1)).astype(jnp.bfloat16)
    pwl_ref[:, 1 * LANES:2 * LANES] = (
        compact(wl1, PlE0) + compact(wr1, PlE1)).astype(jnp.bfloat16)


def _out_kernel(x_ref, wo_ref, bo_ref, o_ref):
    x = x_ref[...].astype(jnp.float32)
    o_ref[...] = lax.dot_general(
        x, wo_ref[...], (((1,), (1,)), ((), ())),
        preferred_element_type=jnp.float32) + bo_ref[...]


def _make_sc_sample():
    mesh = plsc.VectorSubcoreMesh(core_axis_name="c", subcore_axis_name="s")
    H = LANES // 2
    LOCN = (LEN_IN - 5120) * NH

    @functools.partial(
        pl.kernel, mesh=mesh,
        compiler_params=pltpu.CompilerParams(
            use_tc_tiling_on_sc=False, needs_layout_passes=False),
        out_type=jax.ShapeDtypeStruct((ROWS, D), jnp.bfloat16),
        scratch_types=[
            pltpu.VMEM((CH, 4, H), jnp.int32),
            pltpu.VMEM((CH, 4, H), jnp.int32),
            pltpu.VMEM((CH, 4, H), jnp.int32),
            pltpu.VMEM((CH, 4, H), jnp.int32),
            pltpu.VMEM((CH, 2, LANES), jnp.bfloat16),
            pltpu.VMEM((CH, 2, LANES), jnp.bfloat16),
            pltpu.VMEM((CH, 2, LANES), jnp.bfloat16),
            pltpu.VMEM((CH, 2, LANES), jnp.bfloat16),
            pltpu.VMEM((CH, 4, H, HD), jnp.bfloat16),
            pltpu.VMEM((CH, 4, H, HD), jnp.bfloat16),
            pltpu.VMEM((LOCN, HD), jnp.bfloat16),
            pltpu.VMEM((CH, D), jnp.bfloat16),
            pltpu.SemaphoreType.DMA,
            pltpu.SemaphoreType.DMA,
        ],
    )
    def sample(tab, pig, pil, pwg, pwl, out,
               sig0, sig1, sil0, sil1, swg0, swg1, swl0, swl1,
               hrows0, hrows1, ltab, ov, semA, semB):
        wid = lax.axis_index("s") * 2 + lax.axis_index("c")
        base = wid * RW
        NG = RW // CH
        b = wid // (NWORK // B)
        pltpu.sync_copy(
            tab.at[pl.ds(b * (LEN_IN * NH) + 5120 * NH, LOCN)], ltab)

        def load_slab(g, sig, sil, swg, swl):
            r0 = base + g * CH
            pltpu.sync_copy(pig.at[pl.ds(r0, CH)], sig)
            pltpu.sync_copy(pil.at[pl.ds(r0, CH)], sil)
            pltpu.sync_copy(pwg.at[pl.ds(r0, CH)], swg)
            pltpu.sync_copy(pwl.at[pl.ds(r0, CH)], swl)

        def fire(sig, hrows, sem):
            cps = []
            for k in range(CH):
                for c in range(4):
                    cps.append(pltpu.async_copy(
                        tab.at[sig.at[k, c]], hrows.at[k, c], sem))
            return cps

        def compute(g, sil, swg, swl, hrows):
            def body(khh, carry2):
                k = khh // (NH // 2)
                hh = khh - k * (NH // 2)
                hb = hh * 16
                accs = [jnp.zeros((16,), jnp.float32) for _ in range(4)]
                for yc in (0, 1):
                    wv = swg[k, yc, pl.ds(2 * hb, 32)]
                    wl, wr = plsc.unpack(
                        wv, format=plsc.PackFormat.INTERLEAVED)
                    for j in range(16):
                        head = j // 8
                        vl = hrows[k, 2 * yc, hb + j]
                        vr = hrows[k, 2 * yc + 1, hb + j]
                        l0, l1 = plsc.unpack(
                            vl, format=plsc.PackFormat.INTERLEAVED)
                        r0_, r1_ = plsc.unpack(
                            vr, format=plsc.PackFormat.INTERLEAVED)
                        wlv = jnp.broadcast_to(wl[j], (16,))
                        wrv = jnp.broadcast_to(wr[j], (16,))
                        accs[2 * head] = accs[2 * head] + wlv * l0 + wrv * r0_
                        accs[2 * head + 1] = (
                            accs[2 * head + 1] + wlv * l1 + wrv * r1_)
                for yc in (0, 1):
                    il = sil[k, 2 * yc, pl.ds(hb, 16)]
                    ir = sil[k, 2 * yc + 1, pl.ds(hb, 16)]
                    wv = swl[k, yc, pl.ds(2 * hb, 32)]
                    wl, wr = plsc.unpack(
                        wv, format=plsc.PackFormat.INTERLEAVED)
                    for j in range(16):
                        head = j // 8
                        vl = ltab[il[j]]
                        vr = ltab[ir[j]]
                        l0, l1 = plsc.unpack(
                            vl, format=plsc.PackFormat.INTERLEAVED)
                        r0_, r1_ = plsc.unpack(
                            vr, format=plsc.PackFormat.INTERLEAVED)
                        wlv = jnp.broadcast_to(wl[j], (16,))
                        wrv = jnp.broadcast_to(wr[j], (16,))
                        accs[2 * head] = accs[2 * head] + wlv * l0 + wrv * r0_
                        accs[2 * head + 1] = (
                            accs[2 * head + 1] + wlv * l1 + wrv * r1_)
                for head in range(2):
                    hglob = 2 * hh + head
                    ov[k, pl.ds(hglob * HD, HD)] = plsc.pack(
                        accs[2 * head], accs[2 * head + 1],
                        format=plsc.PackFormat.INTERLEAVED)
                return carry2

            lax.fori_loop(0, CH * (NH // 2), body, 0)
            pltpu.sync_copy(ov, out.at[pl.ds(base + g * CH, CH)])

        load_slab(0, sig0, sil0, swg0, swl0)
        load_slab(1, sig1, sil1, swg1, swl1)

        def pair(t, carry):
            g0 = 2 * t
            g1 = 2 * t + 1
            g2 = jnp.minimum(g0 + 2, NG - 1)
            g3 = jnp.minimum(g0 + 3, NG - 1)
            cpsA = fire(sig0, hrows0, semA)
            cpsB = fire(sig1, hrows1, semB)
            for cp in cpsA:
                cp.wait()
            compute(g0, sil0, swg0, swl0, hrows0)
            load_slab(g2, sig0, sil0, swg0, swl0)
            for cp in cpsB:
                cp.wait()
            compute(g1, sil1, swg1, swl1, hrows1)
            load_slab(g3, sig1, sil1, swg1, swl1)
            return carry

        lax.fori_loop(0, NG // 2, pair, 0)

    return sample


_sc_cache = []


def _get_sc_sample():
    if not _sc_cache:
        _sc_cache.append(_make_sc_sample())
    return _sc_cache[0]


def _interleave_perm():
    perm = []
    for h in range(NH):
        for j in range(HD // 2):
            perm.append(h * HD + j)
            perm.append(h * HD + HD // 2 + j)
    return jnp.asarray(perm, jnp.int32)


def kernel(query, reference_points, input_flatten, input_spatial_shapes,
           input_level_start_index, W_so, b_so, W_aw, b_aw, W_v, b_v,
           W_o, b_o):
    q2 = query.reshape(ROWS, D)
    inp2 = input_flatten.reshape(ROWS, D)
    refx = reference_points[..., 0].reshape(ROWS, NL)
    refy = reference_points[..., 1].reshape(ROWS, NL)
    Wx = W_so[0::2]
    Wy = W_so[1::2]
    bx = b_so[0::2].reshape(1, LANES)
    by = b_so[1::2].reshape(1, LANES)
    baw = b_aw.reshape(1, LANES)
    perm = _interleave_perm()
    Wv_p = W_v[perm]
    bv_p = b_v[perm].reshape(1, D)
    bo = b_o.reshape(1, D)

    row_spec = pl.BlockSpec((SBLK, D), lambda i: (i, 0))
    pi_spec = pl.BlockSpec((SBLK, 2 * LANES), lambda i: (i, 0))
    pw_spec = pl.BlockSpec((SBLK, 2 * LANES), lambda i: (i, 0))
    ref_spec = pl.BlockSpec((SBLK, NL), lambda i: (i, 0))

    def full(shape):
        return pl.BlockSpec(shape, lambda i: tuple(0 for _ in shape))

    val, pig, pil, pwg, pwl = pl.pallas_call(
        _proj_kernel,
        grid=(GRID,),
        in_specs=[
            row_spec, row_spec, ref_spec, ref_spec,
            full((LANES, D)), full((LANES, D)), full((LANES, D)),
            full((D, D)),
            full((1, LANES)), full((1, LANES)), full((1, LANES)),
            full((1, D)),
        ],
        out_specs=[row_spec, pi_spec, pi_spec, pw_spec, pw_spec],
        out_shape=[
            jax.ShapeDtypeStruct((ROWS, D), jnp.bfloat16),
            jax.ShapeDtypeStruct((ROWS, 2 * LANES), jnp.int32),
            jax.ShapeDtypeStruct((ROWS, 2 * LANES), jnp.int32),
            jax.ShapeDtypeStruct((ROWS, 2 * LANES), jnp.bfloat16),
            jax.ShapeDtypeStruct((ROWS, 2 * LANES), jnp.bfloat16),
        ],
    )(q2, inp2, refx, refy, Wx, Wy, W_aw, Wv_p, bx, by, baw, bv_p)

    tab = val.reshape(ROWS * NH, HD)
    H = LANES // 2
    sampled = _get_sc_sample()(
        tab,
        pig.reshape(ROWS, 4, H), pil.reshape(ROWS, 4, H),
        pwg.reshape(ROWS, 2, LANES), pwl.reshape(ROWS, 2, LANES))

    Wo_p = W_o[:, perm]
    out = pl.pallas_call(
        _out_kernel,
        grid=(GRID,),
        in_specs=[row_spec, full((D, D)), full((1, D))],
        out_specs=row_spec,
        out_shape=jax.ShapeDtypeStruct((ROWS, D), jnp.float32),
    )(sampled, Wo_p, bo)

    return out.reshape(B, LQ, D)


# speedup vs baseline: 1.0445x; 1.0445x over previous
import functools

import jax
import jax.numpy as jnp
from jax import lax
from jax.experimental import pallas as pl
from jax.experimental.pallas import tpu as pltpu
from jax.experimental.pallas import tpu_sc as plsc

D = 256
NH = 8
NL = 4
NP = 4
HD = D // NH
SPATIAL = ((64, 64), (32, 32), (16, 16), (8, 8))
STARTS = (0, 4096, 5120, 5376)
LEN_IN = 5440
B = 2
LQ = 5440
ROWS = B * LQ
SBLK = 640
GRID = ROWS // SBLK
NWORK = 32
RW = ROWS // NWORK
CH = 5
LANES = NH * NL * NP
NSLAB_I = 2
NSLAB_W = 4


def _proj_kernel(q_ref, inp_ref, refx_ref, refy_ref,
                 wx_ref, wy_ref, waw_ref, wv_ref,
                 bx_ref, by_ref, baw_ref, bv_ref,
                 val_ref, pig_ref, pil_ref, pwg_ref, pwl_ref):
    i = pl.program_id(0)
    q = q_ref[...]

    val = lax.dot_general(
        inp_ref[...], wv_ref[...], (((1,), (1,)), ((), ())),
        preferred_element_type=jnp.float32) + bv_ref[...]
    val_ref[...] = val.astype(jnp.bfloat16)

    sox = lax.dot_general(q, wx_ref[...], (((1,), (1,)), ((), ())),
                          preferred_element_type=jnp.float32) + bx_ref[...]
    soy = lax.dot_general(q, wy_ref[...], (((1,), (1,)), ((), ())),
                          preferred_element_type=jnp.float32) + by_ref[...]

    logit = lax.dot_general(q, waw_ref[...], (((1,), (1,)), ((), ())),
                            preferred_element_type=jnp.float32) + baw_ref[...]
    m = jnp.max(logit, axis=1, keepdims=True)
    e = jnp.exp(logit - m)
    si = lax.broadcasted_iota(jnp.int32, (LANES, LANES), 0)
    sj = lax.broadcasted_iota(jnp.int32, (LANES, LANES), 1)
    seg = ((si >> 4) == (sj >> 4)).astype(jnp.float32)
    denom = lax.dot_general(e, seg, (((1,), (0,)), ((), ())),
                            preferred_element_type=jnp.float32,
                            precision=lax.Precision.HIGHEST)
    aw = e / denom

    li = lax.broadcasted_iota(jnp.int32, (NL, LANES), 0)
    lj = lax.broadcasted_iota(jnp.int32, (NL, LANES), 1)
    exp_mat = (((lj >> 2) & 3) == li).astype(jnp.float32)
    refx = lax.dot_general(refx_ref[...], exp_mat, (((1,), (0,)), ((), ())),
                           preferred_element_type=jnp.float32,
                           precision=lax.Precision.HIGHEST)
    refy = lax.dot_general(refy_ref[...], exp_mat, (((1,), (0,)), ((), ())),
                           preferred_element_type=jnp.float32,
                           precision=lax.Precision.HIGHEST)

    lane = lax.broadcasted_iota(jnp.int32, (SBLK, LANES), 1)
    lvl = (lane >> 2) & 3
    h_lane = lane >> 4

    wi = jnp.full((SBLK, LANES), SPATIAL[0][1], jnp.int32)
    hi = jnp.full((SBLK, LANES), SPATIAL[0][0], jnp.int32)
    st = jnp.full((SBLK, LANES), STARTS[0], jnp.int32)
    for l in range(1, NL):
        wi = jnp.where(lvl == l, SPATIAL[l][1], wi)
        hi = jnp.where(lvl == l, SPATIAL[l][0], hi)
        st = jnp.where(lvl == l, STARTS[l], st)
    wf = wi.astype(jnp.float32)
    hf = hi.astype(jnp.float32)

    x = jnp.clip(refx + sox, 0.0, 1.0) * wf - 0.5
    y = jnp.clip(refy + soy, 0.0, 1.0) * hf - 0.5

    flx = jnp.floor(x).astype(jnp.int32)
    fly = jnp.floor(y).astype(jnp.int32)
    x0 = jnp.clip(flx, 0, wi - 1)
    x1 = jnp.clip(flx + 1, 0, wi - 1)
    y0 = jnp.clip(fly, 0, hi - 1)
    y1 = jnp.clip(fly + 1, 0, hi - 1)
    xa = jnp.clip(flx, 0, wi - 2)
    x0f = x0.astype(jnp.float32)
    x1f = x1.astype(jnp.float32)
    y0f = y0.astype(jnp.float32)
    y1f = y1.astype(jnp.float32)

    wa = aw * ((x1f - x) * (y1f - y))
    wb = aw * ((x1f - x) * (y - y0f))
    wc = aw * ((x - x0f) * (y1f - y))
    wd = aw * ((x - x0f) * (y - y0f))

    zero = jnp.zeros_like(wa)
    sel0 = x0 == xa
    sel1 = x1 == xa
    wl0 = jnp.where(sel0, wa, zero) + jnp.where(sel1, wc, zero)
    wr0 = jnp.where(sel0, zero, wa) + jnp.where(sel1, zero, wc)
    wl1 = jnp.where(sel0, wb, zero) + jnp.where(sel1, wd, zero)
    wr1 = jnp.where(sel0, zero, wb) + jnp.where(sel1, zero, wd)

    row0 = i * SBLK
    ridx = row0 + lax.broadcasted_iota(jnp.int32, (SBLK, LANES), 0)
    base = (ridx // LQ) * (LEN_IN * NH)

    iy0 = base + (st + y0 * wi + xa) * NH + h_lane
    iy1 = base + (st + y1 * wi + xa) * NH + h_lane
    iy0l = ((st - 5120) + y0 * wi + xa) * NH + h_lane
    iy1l = ((st - 5120) + y1 * wi + xa) * NH + h_lane

    ci = lax.broadcasted_iota(jnp.int32, (LANES, LANES // 2), 0)
    cj = lax.broadcasted_iota(jnp.int32, (LANES, LANES // 2), 1)
    clvl = (ci >> 2) & 3
    jg = (ci >> 4) * 8 + ((ci >> 2) & 1) * 4 + (ci & 3)
    jl = (ci >> 4) * 8 + (((ci >> 2) & 3) - 2) * 4 + (ci & 3)
    Pg = ((jg == cj) & (clvl < 2)).astype(jnp.float32)
    Pl = ((jl == cj) & (clvl >= 2)).astype(jnp.float32)

    fi = lax.broadcasted_iota(jnp.int32, (LANES, LANES), 0)
    fj = lax.broadcasted_iota(jnp.int32, (LANES, LANES), 1)
    flvl = (fi >> 2) & 3
    fjg = (fi >> 4) * 8 + ((fi >> 2) & 1) * 4 + (fi & 3)
    fjl = (fi >> 4) * 8 + (((fi >> 2) & 3) - 2) * 4 + (fi & 3)
    PgE0 = ((fj == 2 * fjg) & (flvl < 2)).astype(jnp.float32)
    PgE1 = ((fj == 2 * fjg + 1) & (flvl < 2)).astype(jnp.float32)
    PlE0 = ((fj == 2 * fjl) & (flvl >= 2)).astype(jnp.float32)
    PlE1 = ((fj == 2 * fjl + 1) & (flvl >= 2)).astype(jnp.float32)

    def compact(arr, P):
        return lax.dot_general(arr, P, (((1,), (0,)), ((), ())),
                               preferred_element_type=jnp.float32,
                               precision=lax.Precision.HIGHEST)

    def compact_i(arr, P):
        return compact(arr.astype(jnp.float32), P).astype(jnp.int32)

    H = LANES // 2
    pig_ref[:, 0 * H:1 * H] = compact_i(iy0, Pg)
    pig_ref[:, 1 * H:2 * H] = compact_i(iy0 + NH, Pg)
    pig_ref[:, 2 * H:3 * H] = compact_i(iy1, Pg)
    pig_ref[:, 3 * H:4 * H] = compact_i(iy1 + NH, Pg)
    pil_ref[:, 0 * H:1 * H] = compact_i(iy0l, Pl)
    pil_ref[:, 1 * H:2 * H] = compact_i(iy0l + NH, Pl)
    pil_ref[:, 2 * H:3 * H] = compact_i(iy1l, Pl)
    pil_ref[:, 3 * H:4 * H] = compact_i(iy1l + NH, Pl)
    pwg_ref[:, 0 * LANES:1 * LANES] = (
        compact(wl0, PgE0) + compact(wr0, PgE1)).astype(jnp.bfloat16)
    pwg_ref[:, 1 * LANES:2 * LANES] = (
        compact(wl1, PgE0) + compact(wr1, PgE1)).astype(jnp.bfloat16)
    pwl_ref[:, 0 * LANES:1 * LANES] = (
        compact(wl0, PlE0) + compact(wr0, PlE1)).astype(jnp.bfloat16)
    pwl_ref[:, 1 * LANES:2 * LANES] = (
        compact(wl1, PlE0) + compact(wr1, PlE1)).astype(jnp.bfloat16)


def _out_kernel(x_ref, wo_ref, bo_ref, o_ref):
    o_ref[...] = lax.dot_general(
        x_ref[...], wo_ref[...], (((1,), (1,)), ((), ())),
        preferred_element_type=jnp.float32) + bo_ref[...]


def _make_sc_sample():
    mesh = plsc.VectorSubcoreMesh(core_axis_name="c", subcore_axis_name="s")
    H = LANES // 2
    LOCN = (LEN_IN - 5120) * NH

    @functools.partial(
        pl.kernel, mesh=mesh,
        compiler_params=pltpu.CompilerParams(
            use_tc_tiling_on_sc=False, needs_layout_passes=False),
        out_type=jax.ShapeDtypeStruct((ROWS, D), jnp.float32),
        scratch_types=[
            pltpu.VMEM((CH, 4, H), jnp.int32),
            pltpu.VMEM((CH, 4, H), jnp.int32),
            pltpu.VMEM((CH, 4, H), jnp.int32),
            pltpu.VMEM((CH, 4, H), jnp.int32),
            pltpu.VMEM((CH, 2, LANES), jnp.bfloat16),
            pltpu.VMEM((CH, 2, LANES), jnp.bfloat16),
            pltpu.VMEM((CH, 2, LANES), jnp.bfloat16),
            pltpu.VMEM((CH, 2, LANES), jnp.bfloat16),
            pltpu.VMEM((CH, 4, H, HD), jnp.bfloat16),
            pltpu.VMEM((CH, 4, H, HD), jnp.bfloat16),
            pltpu.VMEM((LOCN, HD), jnp.bfloat16),
            pltpu.VMEM((CH, D), jnp.float32),
            pltpu.SemaphoreType.DMA,
            pltpu.SemaphoreType.DMA,
        ],
    )
    def sample(tab, pig, pil, pwg, pwl, out,
               sig0, sig1, sil0, sil1, swg0, swg1, swl0, swl1,
               hrows0, hrows1, ltab, ov, semA, semB):
        wid = lax.axis_index("s") * 2 + lax.axis_index("c")
        base = wid * RW
        NG = RW // CH
        b = wid // (NWORK // B)
        pltpu.sync_copy(
            tab.at[pl.ds(b * (LEN_IN * NH) + 5120 * NH, LOCN)], ltab)

        def load_slab(g, sig, sil, swg, swl):
            r0 = base + g * CH
            pltpu.sync_copy(pig.at[pl.ds(r0, CH)], sig)
            pltpu.sync_copy(pil.at[pl.ds(r0, CH)], sil)
            pltpu.sync_copy(pwg.at[pl.ds(r0, CH)], swg)
            pltpu.sync_copy(pwl.at[pl.ds(r0, CH)], swl)

        def fire(sig, hrows, sem):
            cps = []
            for k in range(CH):
                for c in range(4):
                    cps.append(pltpu.async_copy(
                        tab.at[sig.at[k, c]], hrows.at[k, c], sem))
            return cps

        def compute(g, sil, swg, swl, hrows):
            def body(khh, carry2):
                k = khh // (NH // 2)
                hh = khh - k * (NH // 2)
                hb = hh * 16
                accs = [jnp.zeros((16,), jnp.float32) for _ in range(4)]
                for yc in (0, 1):
                    wv = swg[k, yc, pl.ds(2 * hb, 32)]
                    wl, wr = plsc.unpack(
                        wv, format=plsc.PackFormat.INTERLEAVED)
                    for j in range(16):
                        head = j // 8
                        vl = hrows[k, 2 * yc, hb + j]
                        vr = hrows[k, 2 * yc + 1, hb + j]
                        l0, l1 = plsc.unpack(
                            vl, format=plsc.PackFormat.INTERLEAVED)
                        r0_, r1_ = plsc.unpack(
                            vr, format=plsc.PackFormat.INTERLEAVED)
                        wlv = jnp.broadcast_to(wl[j], (16,))
                        wrv = jnp.broadcast_to(wr[j], (16,))
                        accs[2 * head] = accs[2 * head] + wlv * l0 + wrv * r0_
                        accs[2 * head + 1] = (
                            accs[2 * head + 1] + wlv * l1 + wrv * r1_)
                for yc in (0, 1):
                    il = sil[k, 2 * yc, pl.ds(hb, 16)]
                    ir = sil[k, 2 * yc + 1, pl.ds(hb, 16)]
                    wv = swl[k, yc, pl.ds(2 * hb, 32)]
                    wl, wr = plsc.unpack(
                        wv, format=plsc.PackFormat.INTERLEAVED)
                    for j in range(16):
                        head = j // 8
                        vl = ltab[il[j]]
                        vr = ltab[ir[j]]
                        l0, l1 = plsc.unpack(
                            vl, format=plsc.PackFormat.INTERLEAVED)
                        r0_, r1_ = plsc.unpack(
                            vr, format=plsc.PackFormat.INTERLEAVED)
                        wlv = jnp.broadcast_to(wl[j], (16,))
                        wrv = jnp.broadcast_to(wr[j], (16,))
                        accs[2 * head] = accs[2 * head] + wlv * l0 + wrv * r0_
                        accs[2 * head + 1] = (
                            accs[2 * head + 1] + wlv * l1 + wrv * r1_)
                for head in range(2):
                    hglob = 2 * hh + head
                    ov[k, pl.ds(hglob * HD, 16)] = accs[2 * head]
                    ov[k, pl.ds(hglob * HD + 16, 16)] = accs[2 * head + 1]
                return carry2

            lax.fori_loop(0, CH * (NH // 2), body, 0)
            pltpu.sync_copy(ov, out.at[pl.ds(base + g * CH, CH)])

        load_slab(0, sig0, sil0, swg0, swl0)
        load_slab(1, sig1, sil1, swg1, swl1)

        def pair(t, carry):
            g0 = 2 * t
            g1 = 2 * t + 1
            g2 = jnp.minimum(g0 + 2, NG - 1)
            g3 = jnp.minimum(g0 + 3, NG - 1)
            cpsA = fire(sig0, hrows0, semA)
            cpsB = fire(sig1, hrows1, semB)
            for cp in cpsA:
                cp.wait()
            compute(g0, sil0, swg0, swl0, hrows0)
            load_slab(g2, sig0, sil0, swg0, swl0)
            for cp in cpsB:
                cp.wait()
            compute(g1, sil1, swg1, swl1, hrows1)
            load_slab(g3, sig1, sil1, swg1, swl1)
            return carry

        lax.fori_loop(0, NG // 2, pair, 0)

    return sample


_sc_cache = []


def _get_sc_sample():
    if not _sc_cache:
        _sc_cache.append(_make_sc_sample())
    return _sc_cache[0]


def _interleave_perm():
    perm = []
    for h in range(NH):
        for j in range(HD // 2):
            perm.append(h * HD + j)
            perm.append(h * HD + HD // 2 + j)
    return jnp.asarray(perm, jnp.int32)


def kernel(query, reference_points, input_flatten, input_spatial_shapes,
           input_level_start_index, W_so, b_so, W_aw, b_aw, W_v, b_v,
           W_o, b_o):
    q2 = query.reshape(ROWS, D)
    inp2 = input_flatten.reshape(ROWS, D)
    refx = reference_points[..., 0].reshape(ROWS, NL)
    refy = reference_points[..., 1].reshape(ROWS, NL)
    Wx = W_so[0::2]
    Wy = W_so[1::2]
    bx = b_so[0::2].reshape(1, LANES)
    by = b_so[1::2].reshape(1, LANES)
    baw = b_aw.reshape(1, LANES)
    perm = _interleave_perm()
    Wv_p = W_v[perm]
    bv_p = b_v[perm].reshape(1, D)
    bo = b_o.reshape(1, D)

    row_spec = pl.BlockSpec((SBLK, D), lambda i: (i, 0))
    pi_spec = pl.BlockSpec((SBLK, 2 * LANES), lambda i: (i, 0))
    pw_spec = pl.BlockSpec((SBLK, 2 * LANES), lambda i: (i, 0))
    ref_spec = pl.BlockSpec((SBLK, NL), lambda i: (i, 0))

    def full(shape):
        return pl.BlockSpec(shape, lambda i: tuple(0 for _ in shape))

    val, pig, pil, pwg, pwl = pl.pallas_call(
        _proj_kernel,
        grid=(GRID,),
        in_specs=[
            row_spec, row_spec, ref_spec, ref_spec,
            full((LANES, D)), full((LANES, D)), full((LANES, D)),
            full((D, D)),
            full((1, LANES)), full((1, LANES)), full((1, LANES)),
            full((1, D)),
        ],
        out_specs=[row_spec, pi_spec, pi_spec, pw_spec, pw_spec],
        out_shape=[
            jax.ShapeDtypeStruct((ROWS, D), jnp.bfloat16),
            jax.ShapeDtypeStruct((ROWS, 2 * LANES), jnp.int32),
            jax.ShapeDtypeStruct((ROWS, 2 * LANES), jnp.int32),
            jax.ShapeDtypeStruct((ROWS, 2 * LANES), jnp.bfloat16),
            jax.ShapeDtypeStruct((ROWS, 2 * LANES), jnp.bfloat16),
        ],
    )(q2, inp2, refx, refy, Wx, Wy, W_aw, Wv_p, bx, by, baw, bv_p)

    tab = val.reshape(ROWS * NH, HD)
    H = LANES // 2
    sampled = _get_sc_sample()(
        tab,
        pig.reshape(ROWS, 4, H), pil.reshape(ROWS, 4, H),
        pwg.reshape(ROWS, 2, LANES), pwl.reshape(ROWS, 2, LANES))

    out = pl.pallas_call(
        _out_kernel,
        grid=(GRID,),
        in_specs=[row_spec, full((D, D)), full((1, D))],
        out_specs=row_spec,
        out_shape=jax.ShapeDtypeStruct((ROWS, D), jnp.float32),
    )(sampled, W_o, bo)

    return out.reshape(B, LQ, D)
